# Initial kernel scaffold; baseline (speedup 1.0000x reference)
import functools
import math

import jax
import jax.numpy as jnp
from jax import lax
from jax.experimental import pallas as pl
from jax.experimental.pallas import tpu as pltpu
from jax.experimental.pallas import tpu_sc as plsc

V = 6890
D = 544
R = 5
A = 8
K3 = 3
RA = R * A
ROT_DELTA = 2
ROTS = tuple(range(0, A, ROT_DELTA))
NROT = len(ROTS)
INV_SQRT = 1.0 / math.sqrt(1.0 + 1e-3)

NW = 32
VP = 6912
NROWS = K3 * RA * VP
ROWS_PER_W = NROWS // NW
CHUNK = 120
NCHUNK = ROWS_PER_W // CHUNK


def _pad16(n):
    return ((n + 15) // 16) * 16


def _sc_gather(table, idx_sc, dpad):
    mesh = plsc.VectorSubcoreMesh(core_axis_name="c", subcore_axis_name="s")

    @functools.partial(
        pl.kernel,
        mesh=mesh,
        out_type=jax.ShapeDtypeStruct((NROWS, dpad), jnp.float32),
        scratch_types=[
            pltpu.VMEM((NCHUNK, CHUNK), jnp.int32),
            pltpu.VMEM((CHUNK, dpad), jnp.float32),
            pltpu.SemaphoreType.DMA,
        ],
    )
    def gather_kernel(idx_hbm, table_hbm, out_hbm, idx_v, rows_v, sem):
        c = lax.axis_index("c")
        s = lax.axis_index("s")
        wid = s * 2 + c
        pltpu.sync_copy(idx_hbm.at[wid], idx_v)
        base0 = wid * ROWS_PER_W

        def body(j, carry):
            pltpu.async_copy(table_hbm.at[idx_v.at[j]], rows_v, sem).wait()
            pltpu.sync_copy(rows_v, out_hbm.at[pl.ds(base0 + j * CHUNK, CHUNK)])
            return carry

        lax.fori_loop(0, NCHUNK, body, 0)

    return gather_kernel(idx_sc, table)


def _down_body(x_ref, w_ref, b_ref, g_ref, bb_ref, o_ref):
    y = jnp.dot(x_ref[...], w_ref[...], preferred_element_type=jnp.float32)
    y = jnp.maximum(y + b_ref[...], 0.0)
    o_ref[...] = g_ref[...] * (y * INV_SQRT) + bb_ref[...]


def _down(signal, w_down, b_down, g, b):
    BV = 576
    grid = VP // BV
    return pl.pallas_call(
        _down_body,
        grid=(grid,),
        in_specs=[
            pl.BlockSpec((BV, D), lambda i: (i, 0)),
            pl.BlockSpec((D, 64), lambda i: (0, 0)),
            pl.BlockSpec((1, 64), lambda i: (0, 0)),
            pl.BlockSpec((1, 64), lambda i: (0, 0)),
            pl.BlockSpec((1, 64), lambda i: (0, 0)),
        ],
        out_specs=pl.BlockSpec((BV, 64), lambda i: (i, 0)),
        out_shape=jax.ShapeDtypeStruct((VP, 64), jnp.float32),
    )(signal, w_down, b_down.reshape(1, 64), g.reshape(1, 64), b.reshape(1, 64))


def _isc_body(rows_ref, w_ref, B_ref, bias_ref, g_ref, bb_ref, o_ref, *, Tp, bn):
    bv = rows_ref.shape[1]
    acc = jnp.zeros((bv, NROT * Tp), jnp.float32)
    for ra in range(RA):
        interp = (w_ref[:, ra:ra + 1] * rows_ref[ra]
                  + w_ref[:, RA + ra:RA + ra + 1] * rows_ref[RA + ra]
                  + w_ref[:, 2 * RA + ra:2 * RA + ra + 1] * rows_ref[2 * RA + ra])
        acc = acc + jnp.dot(interp, B_ref[ra], preferred_element_type=jnp.float32)
    acc = jnp.maximum(acc + bias_ref[...], 0.0)
    m = jnp.maximum(
        jnp.maximum(acc[:, 0 * Tp:1 * Tp], acc[:, 1 * Tp:2 * Tp]),
        jnp.maximum(acc[:, 2 * Tp:3 * Tp], acc[:, 3 * Tp:4 * Tp]),
    )
    if bn:
        m = g_ref[...] * (m * INV_SQRT) + bb_ref[...]
    o_ref[...] = m


def _isc_layer(rows3, w3, Bmat, biasv, g, bb, dpad, Tp, bn):
    BV = 216
    grid = VP // BV
    body = functools.partial(_isc_body, Tp=Tp, bn=bn)
    return pl.pallas_call(
        body,
        grid=(grid,),
        in_specs=[
            pl.BlockSpec((K3 * RA, BV, dpad), lambda i: (0, i, 0)),
            pl.BlockSpec((BV, K3 * RA), lambda i: (i, 0)),
            pl.BlockSpec((RA, dpad, NROT * Tp), lambda i: (0, 0, 0)),
            pl.BlockSpec((1, NROT * Tp), lambda i: (0, 0)),
            pl.BlockSpec((1, Tp), lambda i: (0, 0)),
            pl.BlockSpec((1, Tp), lambda i: (0, 0)),
        ],
        out_specs=pl.BlockSpec((BV, Tp), lambda i: (i, 0)),
        out_shape=jax.ShapeDtypeStruct((VP, Tp), jnp.float32),
    )(rows3, w3, Bmat, biasv, g, bb)


def _final_body(g_ref, l_ref, w_ref, b_ref, o_ref):
    cat = jnp.concatenate([g_ref[:, :50], l_ref[:, :50]], axis=1)
    o_ref[...] = jnp.dot(cat, w_ref[...], preferred_element_type=jnp.float32) + b_ref[...]


def _final(gtab, ltab, w_out, b_out):
    BV = 512
    grid = (V + BV - 1) // BV
    return pl.pallas_call(
        _final_body,
        grid=(grid,),
        in_specs=[
            pl.BlockSpec((BV, 64), lambda i: (i, 0)),
            pl.BlockSpec((BV, 64), lambda i: (i, 0)),
            pl.BlockSpec((100, V), lambda i: (0, 0)),
            pl.BlockSpec((1, V), lambda i: (0, 0)),
        ],
        out_specs=pl.BlockSpec((BV, V), lambda i: (i, 0)),
        out_shape=jax.ShapeDtypeStruct((V, V), jnp.float32),
    )(gtab, ltab, w_out, b_out.reshape(1, V))


def _make_B(t, dpad, Tp):
    T, _, _, d = t.shape
    rolls = jnp.stack([jnp.roll(t, rot, axis=2) for rot in ROTS], axis=0)
    B = jnp.transpose(rolls, (2, 3, 4, 0, 1))
    B = jnp.pad(B, ((0, 0), (0, 0), (0, dpad - d), (0, 0), (0, Tp - T)))
    return B.reshape(RA, dpad, NROT * Tp)


def _pad_vec(x, n):
    return jnp.pad(x, (0, n - x.shape[0]))


def kernel(signal, bc_idx, bc_w, w_down, b_down, bn_down_g, bn_down_b,
           t0, bias0, bn0_g, bn0_b, t1, bias1, bn1_g, bn1_b,
           t2, bias2, bn2_g, bn2_b, t3, bias3, bn3_g, bn3_b,
           t4, bias4, bn4_g, bn4_b, tl, biasl, w_out, b_out):
    idx_t = jnp.transpose(bc_idx, (3, 1, 2, 0)).reshape(K3 * RA, V)
    idx_p = jnp.pad(idx_t, ((0, 0), (0, VP - V)))
    idx_sc = idx_p.reshape(NW, NCHUNK, CHUNK)

    w3 = jnp.transpose(bc_w, (3, 1, 2, 0)).reshape(K3 * RA, V).T
    w3 = jnp.pad(w3, ((0, VP - V), (0, 0)))

    x = _down(signal, w_down, b_down, bn_down_g, bn_down_b)

    layers = [
        (t0, bias0, bn0_g, bn0_b),
        (t1, bias1, bn1_g, bn1_b),
        (t2, bias2, bn2_g, bn2_b),
        (t3, bias3, bn3_g, bn3_b),
        (t4, bias4, bn4_g, bn4_b),
    ]
    g = x
    for t, bi, bg, bb in layers:
        T, _, _, d = t.shape
        dpad = _pad16(d)
        Tp = _pad16(T)
        Bmat = _make_B(t, dpad, Tp)
        biasv = jnp.tile(_pad_vec(bi, Tp), (NROT,)).reshape(1, NROT * Tp)
        gv = _pad_vec(bg, Tp).reshape(1, Tp)
        bv = _pad_vec(bb, Tp).reshape(1, Tp)
        gpad = g if g.shape[1] == dpad else jnp.pad(g, ((0, 0), (0, dpad - g.shape[1])))
        rows = _sc_gather(gpad, idx_sc, dpad)
        rows3 = rows.reshape(K3 * RA, VP, dpad)
        g = _isc_layer(rows3, w3, Bmat, biasv, gv, bv, dpad, Tp, True)

    T, _, _, d = tl.shape
    dpad = _pad16(d)
    Tp = _pad16(T)
    Bl = _make_B(tl, dpad, Tp)
    biaslv = jnp.tile(_pad_vec(biasl, Tp), (NROT,)).reshape(1, NROT * Tp)
    zeros = jnp.zeros((1, Tp), jnp.float32)
    rowsl = _sc_gather(x, idx_sc, dpad)
    rows3l = rowsl.reshape(K3 * RA, VP, dpad)
    ltab = _isc_layer(rows3l, w3, Bl, biaslv, zeros, zeros, dpad, Tp, False)

    return _final(g, ltab, w_out, b_out)


# speedup vs baseline: 4.3720x; 4.3720x over previous
import functools
import math

import jax
import jax.numpy as jnp
from jax import lax
from jax.experimental import pallas as pl
from jax.experimental.pallas import tpu as pltpu
from jax.experimental.pallas import tpu_sc as plsc

V = 6890
D = 544
R = 5
A = 8
K3 = 3
RA = R * A
ROT_DELTA = 2
ROTS = tuple(range(0, A, ROT_DELTA))
NROT = len(ROTS)
INV_SQRT = 1.0 / math.sqrt(1.0 + 1e-3)

NW = 32
VP = 6912
NROWS = K3 * RA * VP
ROWS_PER_W = NROWS // NW
CHUNK = 120
NCHUNK = ROWS_PER_W // CHUNK


def _pad16(n):
    return ((n + 15) // 16) * 16


def _sc_gather(table, idx_sc, dpad):
    mesh = plsc.VectorSubcoreMesh(core_axis_name="c", subcore_axis_name="s")

    @functools.partial(
        pl.kernel,
        mesh=mesh,
        out_type=jax.ShapeDtypeStruct((NROWS, dpad), jnp.float32),
        scratch_types=[
            pltpu.VMEM((NCHUNK, CHUNK), jnp.int32),
            pltpu.VMEM((CHUNK, dpad), jnp.float32),
            pltpu.SemaphoreType.DMA,
        ],
        compiler_params=pltpu.CompilerParams(use_tc_tiling_on_sc=False),
    )
    def gather_kernel(idx_hbm, table_hbm, out_hbm, idx_v, rows_v, sem):
        c = lax.axis_index("c")
        s = lax.axis_index("s")
        wid = s * 2 + c
        pltpu.sync_copy(idx_hbm.at[wid], idx_v)
        base0 = wid * ROWS_PER_W

        def body(j, carry):
            pltpu.async_copy(table_hbm.at[idx_v.at[j]], rows_v, sem).wait()
            pltpu.sync_copy(rows_v, out_hbm.at[pl.ds(base0 + j * CHUNK, CHUNK)])
            return carry

        lax.fori_loop(0, NCHUNK, body, 0)

    return gather_kernel(idx_sc, table)


def _down_body(x_ref, w_ref, b_ref, g_ref, bb_ref, o_ref):
    y = jnp.dot(x_ref[...], w_ref[...], preferred_element_type=jnp.float32)
    y = jnp.maximum(y + b_ref[...], 0.0)
    o_ref[...] = g_ref[...] * (y * INV_SQRT) + bb_ref[...]


def _down(signal, w_down, b_down, g, b):
    BV = 576
    grid = VP // BV
    return pl.pallas_call(
        _down_body,
        grid=(grid,),
        in_specs=[
            pl.BlockSpec((BV, D), lambda i: (i, 0)),
            pl.BlockSpec((D, 64), lambda i: (0, 0)),
            pl.BlockSpec((1, 64), lambda i: (0, 0)),
            pl.BlockSpec((1, 64), lambda i: (0, 0)),
            pl.BlockSpec((1, 64), lambda i: (0, 0)),
        ],
        out_specs=pl.BlockSpec((BV, 64), lambda i: (i, 0)),
        out_shape=jax.ShapeDtypeStruct((VP, 64), jnp.float32),
    )(signal, w_down, b_down.reshape(1, 64), g.reshape(1, 64), b.reshape(1, 64))


def _isc_body(rows_ref, w_ref, B_ref, bias_ref, g_ref, bb_ref, o_ref, *, Tp, bn):
    bv = rows_ref.shape[1]
    acc = jnp.zeros((bv, NROT * Tp), jnp.float32)
    for ra in range(RA):
        interp = (w_ref[:, ra:ra + 1] * rows_ref[ra]
                  + w_ref[:, RA + ra:RA + ra + 1] * rows_ref[RA + ra]
                  + w_ref[:, 2 * RA + ra:2 * RA + ra + 1] * rows_ref[2 * RA + ra])
        acc = acc + jnp.dot(interp, B_ref[ra], preferred_element_type=jnp.float32)
    acc = jnp.maximum(acc + bias_ref[...], 0.0)
    m = jnp.maximum(
        jnp.maximum(acc[:, 0 * Tp:1 * Tp], acc[:, 1 * Tp:2 * Tp]),
        jnp.maximum(acc[:, 2 * Tp:3 * Tp], acc[:, 3 * Tp:4 * Tp]),
    )
    if bn:
        m = g_ref[...] * (m * INV_SQRT) + bb_ref[...]
    o_ref[...] = m


def _isc_layer(rows3, w3, Bmat, biasv, g, bb, dpad, Tp, bn):
    BV = 216
    grid = VP // BV
    body = functools.partial(_isc_body, Tp=Tp, bn=bn)
    return pl.pallas_call(
        body,
        grid=(grid,),
        in_specs=[
            pl.BlockSpec((K3 * RA, BV, dpad), lambda i: (0, i, 0)),
            pl.BlockSpec((BV, K3 * RA), lambda i: (i, 0)),
            pl.BlockSpec((RA, dpad, NROT * Tp), lambda i: (0, 0, 0)),
            pl.BlockSpec((1, NROT * Tp), lambda i: (0, 0)),
            pl.BlockSpec((1, Tp), lambda i: (0, 0)),
            pl.BlockSpec((1, Tp), lambda i: (0, 0)),
        ],
        out_specs=pl.BlockSpec((BV, Tp), lambda i: (i, 0)),
        out_shape=jax.ShapeDtypeStruct((VP, Tp), jnp.float32),
    )(rows3, w3, Bmat, biasv, g, bb)


def _final_body(g_ref, l_ref, w_ref, b_ref, o_ref):
    cat = jnp.concatenate([g_ref[:, :50], l_ref[:, :50]], axis=1)
    o_ref[...] = jnp.dot(cat, w_ref[...], preferred_element_type=jnp.float32) + b_ref[...]


def _final(gtab, ltab, w_out, b_out):
    BV = 512
    grid = (V + BV - 1) // BV
    return pl.pallas_call(
        _final_body,
        grid=(grid,),
        in_specs=[
            pl.BlockSpec((BV, 64), lambda i: (i, 0)),
            pl.BlockSpec((BV, 64), lambda i: (i, 0)),
            pl.BlockSpec((100, V), lambda i: (0, 0)),
            pl.BlockSpec((1, V), lambda i: (0, 0)),
        ],
        out_specs=pl.BlockSpec((BV, V), lambda i: (i, 0)),
        out_shape=jax.ShapeDtypeStruct((V, V), jnp.float32),
    )(gtab, ltab, w_out, b_out.reshape(1, V))


def _make_B(t, dpad, Tp):
    T, _, _, d = t.shape
    rolls = jnp.stack([jnp.roll(t, rot, axis=2) for rot in ROTS], axis=0)
    B = jnp.transpose(rolls, (2, 3, 4, 0, 1))
    B = jnp.pad(B, ((0, 0), (0, 0), (0, dpad - d), (0, 0), (0, Tp - T)))
    return B.reshape(RA, dpad, NROT * Tp)


def _pad_vec(x, n):
    return jnp.pad(x, (0, n - x.shape[0]))


def kernel(signal, bc_idx, bc_w, w_down, b_down, bn_down_g, bn_down_b,
           t0, bias0, bn0_g, bn0_b, t1, bias1, bn1_g, bn1_b,
           t2, bias2, bn2_g, bn2_b, t3, bias3, bn3_g, bn3_b,
           t4, bias4, bn4_g, bn4_b, tl, biasl, w_out, b_out):
    idx_t = jnp.transpose(bc_idx, (3, 1, 2, 0)).reshape(K3 * RA, V)
    idx_p = jnp.pad(idx_t, ((0, 0), (0, VP - V)))
    idx_sc = idx_p.reshape(NW, NCHUNK, CHUNK)

    w3 = jnp.transpose(bc_w, (3, 1, 2, 0)).reshape(K3 * RA, V).T
    w3 = jnp.pad(w3, ((0, VP - V), (0, 0)))

    x = _down(signal, w_down, b_down, bn_down_g, bn_down_b)

    layers = [
        (t0, bias0, bn0_g, bn0_b),
        (t1, bias1, bn1_g, bn1_b),
        (t2, bias2, bn2_g, bn2_b),
        (t3, bias3, bn3_g, bn3_b),
        (t4, bias4, bn4_g, bn4_b),
    ]
    g = x
    for t, bi, bg, bb in layers:
        T, _, _, d = t.shape
        dpad = _pad16(d)
        Tp = _pad16(T)
        Bmat = _make_B(t, dpad, Tp)
        biasv = jnp.tile(_pad_vec(bi, Tp), (NROT,)).reshape(1, NROT * Tp)
        gv = _pad_vec(bg, Tp).reshape(1, Tp)
        bv = _pad_vec(bb, Tp).reshape(1, Tp)
        gpad = g if g.shape[1] == dpad else jnp.pad(g, ((0, 0), (0, dpad - g.shape[1])))
        rows = _sc_gather(gpad, idx_sc, dpad)
        rows3 = rows.reshape(K3 * RA, VP, dpad)
        g = _isc_layer(rows3, w3, Bmat, biasv, gv, bv, dpad, Tp, True)

    T, _, _, d = tl.shape
    dpad = _pad16(d)
    Tp = _pad16(T)
    Bl = _make_B(tl, dpad, Tp)
    biaslv = jnp.tile(_pad_vec(biasl, Tp), (NROT,)).reshape(1, NROT * Tp)
    zeros = jnp.zeros((1, Tp), jnp.float32)
    rowsl = _sc_gather(x, idx_sc, dpad)
    rows3l = rowsl.reshape(K3 * RA, VP, dpad)
    ltab = _isc_layer(rows3l, w3, Bl, biaslv, zeros, zeros, dpad, Tp, False)

    return _final(g, ltab, w_out, b_out)


# speedup vs baseline: 7.7137x; 1.7644x over previous
import functools
import math

import jax
import jax.numpy as jnp
from jax import lax
from jax.experimental import pallas as pl
from jax.experimental.pallas import tpu as pltpu
from jax.experimental.pallas import tpu_sc as plsc

V = 6890
D = 544
R = 5
A = 8
K3 = 3
RA = R * A
C120 = K3 * RA
ROT_DELTA = 2
ROTS = tuple(range(0, A, ROT_DELTA))
NROT = len(ROTS)
INV_SQRT = 1.0 / math.sqrt(1.0 + 1e-3)

NW = 32
VP = 6912
VW = VP // NW
CH = 54
NC_V = VW // CH
NBUF = 6


def _pad16(n):
    return ((n + 15) // 16) * 16


def _sc_gather(table, idx_sc, dpad):
    P = 128 // dpad
    NQ = C120 // P
    NU = NQ * NC_V
    NG = NU // NBUF
    assert NG % 2 == 0 and NG * NBUF == NU
    mesh = plsc.VectorSubcoreMesh(core_axis_name="c", subcore_axis_name="s")

    @functools.partial(
        pl.kernel,
        mesh=mesh,
        out_type=jax.ShapeDtypeStruct((NQ, VP, 128), jnp.float32),
        scratch_types=[
            pltpu.VMEM((NU * P, CH), jnp.int32),
            pltpu.VMEM((2 * NBUF, P, CH, dpad), jnp.float32),
            pltpu.SemaphoreType.DMA((2 * NBUF,)),
            pltpu.SemaphoreType.DMA((2 * NBUF,)),
        ],
        compiler_params=pltpu.CompilerParams(use_tc_tiling_on_sc=False),
    )
    def gather_kernel(idx_hbm, table_hbm, out_hbm, idx_v, rows_v, gsem, osem):
        c = lax.axis_index("c")
        s = lax.axis_index("s")
        wid = s * 2 + c
        pltpu.sync_copy(idx_hbm.at[wid], idx_v)
        vbase0 = wid * VW

        def gathers(u, slot, wait):
            for h in range(P):
                cp = pltpu.make_async_copy(
                    table_hbm.at[idx_v.at[u * P + h]],
                    rows_v.at[slot, h],
                    gsem.at[slot],
                )
                if wait:
                    cp.wait()
                else:
                    cp.start()

        def scatter(u, slot, wait):
            q = u // NC_V
            vb = vbase0 + (u % NC_V) * CH
            for h in range(P):
                cp = pltpu.make_async_copy(
                    rows_v.at[slot, h],
                    out_hbm.at[q, pl.ds(vb, CH), pl.ds(h * dpad, dpad)],
                    osem.at[slot],
                )
                if wait:
                    cp.wait()
                else:
                    cp.start()

        for b in range(NBUF):
            gathers(b, b, False)

        def group(g, base):
            for b in range(NBUF):
                u = g * NBUF + b
                gathers(u, base + b, True)
                scatter(u, base + b, False)
            other = NBUF - base

            @pl.when(g >= 1)
            def _():
                for b in range(NBUF):
                    scatter((g - 1) * NBUF + b, other + b, True)

            @pl.when(g + 1 < NG)
            def _():
                for b in range(NBUF):
                    gathers((g + 1) * NBUF + b, other + b, False)

        def pair(g2, carry):
            group(2 * g2, 0)
            group(2 * g2 + 1, NBUF)
            return carry

        lax.fori_loop(0, NG // 2, pair, 0)
        for b in range(NBUF):
            scatter((NG - 1) * NBUF + b, NBUF + b, True)

    return gather_kernel(idx_sc, table)


def _down_body(x_ref, w_ref, b_ref, g_ref, bb_ref, o_ref):
    y = jnp.dot(x_ref[...], w_ref[...], preferred_element_type=jnp.float32)
    y = jnp.maximum(y + b_ref[...], 0.0)
    o_ref[...] = g_ref[...] * (y * INV_SQRT) + bb_ref[...]


def _down(signal, w_down, b_down, g, b):
    BV = 576
    grid = VP // BV
    return pl.pallas_call(
        _down_body,
        grid=(grid,),
        in_specs=[
            pl.BlockSpec((BV, D), lambda i: (i, 0)),
            pl.BlockSpec((D, 64), lambda i: (0, 0)),
            pl.BlockSpec((1, 64), lambda i: (0, 0)),
            pl.BlockSpec((1, 64), lambda i: (0, 0)),
            pl.BlockSpec((1, 64), lambda i: (0, 0)),
        ],
        out_specs=pl.BlockSpec((BV, 64), lambda i: (i, 0)),
        out_shape=jax.ShapeDtypeStruct((VP, 64), jnp.float32),
    )(signal, w_down, b_down.reshape(1, 64), g.reshape(1, 64), b.reshape(1, 64))


def _isc_body(rows_ref, w_ref, B_ref, bias_ref, g_ref, bb_ref, o_ref, *,
              dpad, Tp, bn):
    P = 128 // dpad
    bv = rows_ref.shape[1]

    def rowslab(cc):
        return rows_ref[cc // P, :, (cc % P) * dpad:(cc % P + 1) * dpad]

    acc = jnp.zeros((bv, NROT * Tp), jnp.float32)
    for ra in range(RA):
        interp = (w_ref[:, ra:ra + 1] * rowslab(ra)
                  + w_ref[:, RA + ra:RA + ra + 1] * rowslab(RA + ra)
                  + w_ref[:, 2 * RA + ra:2 * RA + ra + 1] * rowslab(2 * RA + ra))
        acc = acc + jnp.dot(interp, B_ref[ra], preferred_element_type=jnp.float32)
    acc = jnp.maximum(acc + bias_ref[...], 0.0)
    m = jnp.maximum(
        jnp.maximum(acc[:, 0 * Tp:1 * Tp], acc[:, 1 * Tp:2 * Tp]),
        jnp.maximum(acc[:, 2 * Tp:3 * Tp], acc[:, 3 * Tp:4 * Tp]),
    )
    if bn:
        m = g_ref[...] * (m * INV_SQRT) + bb_ref[...]
    o_ref[...] = m


def _isc_layer(rows3, w3, Bmat, biasv, g, bb, dpad, Tp, bn):
    BV = 216
    grid = VP // BV
    NQ = rows3.shape[0]
    body = functools.partial(_isc_body, dpad=dpad, Tp=Tp, bn=bn)
    return pl.pallas_call(
        body,
        grid=(grid,),
        in_specs=[
            pl.BlockSpec((NQ, BV, 128), lambda i: (0, i, 0)),
            pl.BlockSpec((BV, C120), lambda i: (i, 0)),
            pl.BlockSpec((RA, dpad, NROT * Tp), lambda i: (0, 0, 0)),
            pl.BlockSpec((1, NROT * Tp), lambda i: (0, 0)),
            pl.BlockSpec((1, Tp), lambda i: (0, 0)),
            pl.BlockSpec((1, Tp), lambda i: (0, 0)),
        ],
        out_specs=pl.BlockSpec((BV, Tp), lambda i: (i, 0)),
        out_shape=jax.ShapeDtypeStruct((VP, Tp), jnp.float32),
    )(rows3, w3, Bmat, biasv, g, bb)


def _final_body(g_ref, l_ref, w_ref, b_ref, o_ref):
    cat = jnp.concatenate([g_ref[:, :50], l_ref[:, :50]], axis=1)
    o_ref[...] = jnp.dot(cat, w_ref[...], preferred_element_type=jnp.float32) + b_ref[...]


def _final(gtab, ltab, w_out, b_out):
    BV = 512
    grid = (V + BV - 1) // BV
    return pl.pallas_call(
        _final_body,
        grid=(grid,),
        in_specs=[
            pl.BlockSpec((BV, 64), lambda i: (i, 0)),
            pl.BlockSpec((BV, 64), lambda i: (i, 0)),
            pl.BlockSpec((100, V), lambda i: (0, 0)),
            pl.BlockSpec((1, V), lambda i: (0, 0)),
        ],
        out_specs=pl.BlockSpec((BV, V), lambda i: (i, 0)),
        out_shape=jax.ShapeDtypeStruct((V, V), jnp.float32),
    )(gtab, ltab, w_out, b_out.reshape(1, V))


def _make_B(t, dpad, Tp):
    T, _, _, d = t.shape
    rolls = jnp.stack([jnp.roll(t, rot, axis=2) for rot in ROTS], axis=0)
    B = jnp.transpose(rolls, (2, 3, 4, 0, 1))
    B = jnp.pad(B, ((0, 0), (0, 0), (0, dpad - d), (0, 0), (0, Tp - T)))
    return B.reshape(RA, dpad, NROT * Tp)


def _pack_idx(idx_c, dpad):
    P = 128 // dpad
    NQ = C120 // P
    x = idx_c.reshape(NQ, P, NW, NC_V, CH)
    x = jnp.transpose(x, (2, 0, 3, 1, 4))
    return x.reshape(NW, NQ * NC_V * P, CH)


def _pad_vec(x, n):
    return jnp.pad(x, (0, n - x.shape[0]))


def kernel(signal, bc_idx, bc_w, w_down, b_down, bn_down_g, bn_down_b,
           t0, bias0, bn0_g, bn0_b, t1, bias1, bn1_g, bn1_b,
           t2, bias2, bn2_g, bn2_b, t3, bias3, bn3_g, bn3_b,
           t4, bias4, bn4_g, bn4_b, tl, biasl, w_out, b_out):
    idx_t = jnp.transpose(bc_idx, (3, 1, 2, 0)).reshape(C120, V)
    idx_c = jnp.pad(idx_t, ((0, 0), (0, VP - V)))
    idx_by_dpad = {dp: _pack_idx(idx_c, dp) for dp in (64, 32, 16)}

    w3 = jnp.transpose(bc_w, (3, 1, 2, 0)).reshape(C120, V).T
    w3 = jnp.pad(w3, ((0, VP - V), (0, 0)))

    x = _down(signal, w_down, b_down, bn_down_g, bn_down_b)

    layers = [
        (t0, bias0, bn0_g, bn0_b),
        (t1, bias1, bn1_g, bn1_b),
        (t2, bias2, bn2_g, bn2_b),
        (t3, bias3, bn3_g, bn3_b),
        (t4, bias4, bn4_g, bn4_b),
    ]
    g = x
    for t, bi, bg, bb in layers:
        T, _, _, d = t.shape
        dpad = _pad16(d)
        Tp = _pad16(T)
        Bmat = _make_B(t, dpad, Tp)
        biasv = jnp.tile(_pad_vec(bi, Tp), (NROT,)).reshape(1, NROT * Tp)
        gv = _pad_vec(bg, Tp).reshape(1, Tp)
        bv = _pad_vec(bb, Tp).reshape(1, Tp)
        gpad = g if g.shape[1] == dpad else jnp.pad(g, ((0, 0), (0, dpad - g.shape[1])))
        rows3 = _sc_gather(gpad, idx_by_dpad[dpad], dpad)
        g = _isc_layer(rows3, w3, Bmat, biasv, gv, bv, dpad, Tp, True)

    T, _, _, d = tl.shape
    dpad = _pad16(d)
    Tp = _pad16(T)
    Bl = _make_B(tl, dpad, Tp)
    biaslv = jnp.tile(_pad_vec(biasl, Tp), (NROT,)).reshape(1, NROT * Tp)
    zeros = jnp.zeros((1, Tp), jnp.float32)
    rows3l = _sc_gather(x, idx_by_dpad[dpad], dpad)
    ltab = _isc_layer(rows3l, w3, Bl, biaslv, zeros, zeros, dpad, Tp, False)

    return _final(g, ltab, w_out, b_out)


# speedup vs baseline: 11.4154x; 1.4799x over previous
import functools
import math

import jax
import jax.numpy as jnp
from jax import lax
from jax.experimental import pallas as pl
from jax.experimental.pallas import tpu as pltpu
from jax.experimental.pallas import tpu_sc as plsc

V = 6890
D = 544
R = 5
A = 8
K3 = 3
RA = R * A
C120 = K3 * RA
ROT_DELTA = 2
ROTS = tuple(range(0, A, ROT_DELTA))
NROT = len(ROTS)
INV_SQRT = 1.0 / math.sqrt(1.0 + 1e-3)

NW = 32
VP = 6912
VW = VP // NW
CH = 54
NC_V = VW // CH
NBUF = 6


def _pad16(n):
    return ((n + 15) // 16) * 16


def _sc_gather(table, idx_sc, dpad):
    P = 128 // dpad
    NQ = C120 // P
    NU = NQ * NC_V
    NG = NU // NBUF
    assert NG % 2 == 0 and NG * NBUF == NU
    mesh = plsc.VectorSubcoreMesh(core_axis_name="c", subcore_axis_name="s")

    @functools.partial(
        pl.kernel,
        mesh=mesh,
        out_type=jax.ShapeDtypeStruct((NQ, VP, 128), jnp.float32),
        scratch_types=[
            pltpu.VMEM((NU * P, CH), jnp.int32),
            pltpu.VMEM((2 * NBUF, P, CH, dpad), jnp.float32),
            pltpu.SemaphoreType.DMA((2 * NBUF,)),
            pltpu.SemaphoreType.DMA((2 * NBUF,)),
        ],
        compiler_params=pltpu.CompilerParams(use_tc_tiling_on_sc=False),
    )
    def gather_kernel(idx_hbm, table_hbm, out_hbm, idx_v, rows_v, gsem, osem):
        c = lax.axis_index("c")
        s = lax.axis_index("s")
        wid = s * 2 + c
        pltpu.sync_copy(idx_hbm.at[wid], idx_v)
        vbase0 = wid * VW

        def gathers(u, slot, wait):
            for h in range(P):
                cp = pltpu.make_async_copy(
                    table_hbm.at[idx_v.at[u * P + h]],
                    rows_v.at[slot, h],
                    gsem.at[slot],
                )
                if wait:
                    cp.wait()
                else:
                    cp.start()

        def scatter(u, slot, wait):
            q = u // NC_V
            vb = vbase0 + (u % NC_V) * CH
            for h in range(P):
                cp = pltpu.make_async_copy(
                    rows_v.at[slot, h],
                    out_hbm.at[q, pl.ds(vb, CH), pl.ds(h * dpad, dpad)],
                    osem.at[slot],
                )
                if wait:
                    cp.wait()
                else:
                    cp.start()

        for b in range(NBUF):
            gathers(b, b, False)

        def group(g, base):
            for b in range(NBUF):
                u = g * NBUF + b
                gathers(u, base + b, True)
                scatter(u, base + b, False)
            other = NBUF - base

            @pl.when(g >= 1)
            def _():
                for b in range(NBUF):
                    scatter((g - 1) * NBUF + b, other + b, True)

            @pl.when(g + 1 < NG)
            def _():
                for b in range(NBUF):
                    gathers((g + 1) * NBUF + b, other + b, False)

        def pair(g2, carry):
            group(2 * g2, 0)
            group(2 * g2 + 1, NBUF)
            return carry

        lax.fori_loop(0, NG // 2, pair, 0)
        for b in range(NBUF):
            scatter((NG - 1) * NBUF + b, NBUF + b, True)

    return gather_kernel(idx_sc, table)


def _down_body(x_ref, w_ref, b_ref, g_ref, bb_ref, o_ref):
    y = jnp.dot(x_ref[...], w_ref[...], preferred_element_type=jnp.float32)
    y = jnp.maximum(y + b_ref[...], 0.0)
    o_ref[...] = g_ref[...] * (y * INV_SQRT) + bb_ref[...]


def _down(signal, w_down, b_down, g, b):
    BV = 576
    grid = VP // BV
    return pl.pallas_call(
        _down_body,
        grid=(grid,),
        in_specs=[
            pl.BlockSpec((BV, D), lambda i: (i, 0)),
            pl.BlockSpec((D, 64), lambda i: (0, 0)),
            pl.BlockSpec((1, 64), lambda i: (0, 0)),
            pl.BlockSpec((1, 64), lambda i: (0, 0)),
            pl.BlockSpec((1, 64), lambda i: (0, 0)),
        ],
        out_specs=pl.BlockSpec((BV, 64), lambda i: (i, 0)),
        out_shape=jax.ShapeDtypeStruct((VP, 64), jnp.float32),
    )(signal, w_down, b_down.reshape(1, 64), g.reshape(1, 64), b.reshape(1, 64))


def _isc_body(rows_ref, w_ref, e_ref, B_ref, bias_ref, g_ref, bb_ref, o_ref, *,
              dpad, Tp, bn):
    P = 128 // dpad
    NQ = C120 // P
    NK = RA // P
    bv = rows_ref.shape[1]

    wrows = []
    for q in range(NQ):
        wb = jnp.dot(w_ref[:, q * P:(q + 1) * P], e_ref[...],
                     preferred_element_type=jnp.float32)
        wrows.append(wb * rows_ref[q])
    interp = [wrows[qp] + wrows[NK + qp] + wrows[2 * NK + qp]
              for qp in range(NK)]

    acc = jnp.zeros((bv, NROT * Tp), jnp.float32)
    if NK % 2 == 0:
        for j in range(NK // 2):
            blk = jnp.concatenate([interp[2 * j], interp[2 * j + 1]], axis=1)
            acc = acc + jnp.dot(blk, B_ref[j], preferred_element_type=jnp.float32)
    else:
        for j in range(NK):
            acc = acc + jnp.dot(interp[j], B_ref[j], preferred_element_type=jnp.float32)
    acc = jnp.maximum(acc + bias_ref[...], 0.0)
    m = jnp.maximum(
        jnp.maximum(acc[:, 0 * Tp:1 * Tp], acc[:, 1 * Tp:2 * Tp]),
        jnp.maximum(acc[:, 2 * Tp:3 * Tp], acc[:, 3 * Tp:4 * Tp]),
    )
    if bn:
        m = g_ref[...] * (m * INV_SQRT) + bb_ref[...]
    o_ref[...] = m


def _isc_layer(rows3, w3, Bmat, biasv, g, bb, dpad, Tp, bn):
    BV = 216
    grid = VP // BV
    NQ = rows3.shape[0]
    P = 128 // dpad
    NK = RA // P
    NB = NK // 2 if NK % 2 == 0 else NK
    Bmat = Bmat.reshape(NB, -1, NROT * Tp)
    emat = jnp.repeat(jnp.eye(P, dtype=jnp.float32), dpad, axis=1)
    body = functools.partial(_isc_body, dpad=dpad, Tp=Tp, bn=bn)
    return pl.pallas_call(
        body,
        grid=(grid,),
        in_specs=[
            pl.BlockSpec((NQ, BV, 128), lambda i: (0, i, 0)),
            pl.BlockSpec((BV, C120), lambda i: (i, 0)),
            pl.BlockSpec((P, 128), lambda i: (0, 0)),
            pl.BlockSpec(Bmat.shape, lambda i: (0, 0, 0)),
            pl.BlockSpec((1, NROT * Tp), lambda i: (0, 0)),
            pl.BlockSpec((1, Tp), lambda i: (0, 0)),
            pl.BlockSpec((1, Tp), lambda i: (0, 0)),
        ],
        out_specs=pl.BlockSpec((BV, Tp), lambda i: (i, 0)),
        out_shape=jax.ShapeDtypeStruct((VP, Tp), jnp.float32),
    )(rows3, w3, emat, Bmat, biasv, g, bb)


def _final_body(g_ref, l_ref, w_ref, b_ref, o_ref):
    cat = jnp.concatenate([g_ref[:, :50], l_ref[:, :50]], axis=1)
    o_ref[...] = jnp.dot(cat, w_ref[...], preferred_element_type=jnp.float32) + b_ref[...]


def _final(gtab, ltab, w_out, b_out):
    BV = 512
    grid = (V + BV - 1) // BV
    return pl.pallas_call(
        _final_body,
        grid=(grid,),
        in_specs=[
            pl.BlockSpec((BV, 64), lambda i: (i, 0)),
            pl.BlockSpec((BV, 64), lambda i: (i, 0)),
            pl.BlockSpec((100, V), lambda i: (0, 0)),
            pl.BlockSpec((1, V), lambda i: (0, 0)),
        ],
        out_specs=pl.BlockSpec((BV, V), lambda i: (i, 0)),
        out_shape=jax.ShapeDtypeStruct((V, V), jnp.float32),
    )(gtab, ltab, w_out, b_out.reshape(1, V))


def _make_B(t, dpad, Tp):
    T, _, _, d = t.shape
    rolls = jnp.stack([jnp.roll(t, rot, axis=2) for rot in ROTS], axis=0)
    B = jnp.transpose(rolls, (2, 3, 4, 0, 1))
    B = jnp.pad(B, ((0, 0), (0, 0), (0, dpad - d), (0, 0), (0, Tp - T)))
    return B.reshape(RA, dpad, NROT * Tp)


def _pack_idx(idx_c, dpad):
    P = 128 // dpad
    NQ = C120 // P
    x = idx_c.reshape(NQ, P, NW, NC_V, CH)
    x = jnp.transpose(x, (2, 0, 3, 1, 4))
    return x.reshape(NW, NQ * NC_V * P, CH)


def _pad_vec(x, n):
    return jnp.pad(x, (0, n - x.shape[0]))


def kernel(signal, bc_idx, bc_w, w_down, b_down, bn_down_g, bn_down_b,
           t0, bias0, bn0_g, bn0_b, t1, bias1, bn1_g, bn1_b,
           t2, bias2, bn2_g, bn2_b, t3, bias3, bn3_g, bn3_b,
           t4, bias4, bn4_g, bn4_b, tl, biasl, w_out, b_out):
    idx_t = jnp.transpose(bc_idx, (3, 1, 2, 0)).reshape(C120, V)
    idx_c = jnp.pad(idx_t, ((0, 0), (0, VP - V)))
    idx_by_dpad = {dp: _pack_idx(idx_c, dp) for dp in (64, 32, 16)}

    w3 = jnp.transpose(bc_w, (3, 1, 2, 0)).reshape(C120, V).T
    w3 = jnp.pad(w3, ((0, VP - V), (0, 0)))

    x = _down(signal, w_down, b_down, bn_down_g, bn_down_b)

    layers = [
        (t0, bias0, bn0_g, bn0_b),
        (t1, bias1, bn1_g, bn1_b),
        (t2, bias2, bn2_g, bn2_b),
        (t3, bias3, bn3_g, bn3_b),
        (t4, bias4, bn4_g, bn4_b),
    ]
    g = x
    for t, bi, bg, bb in layers:
        T, _, _, d = t.shape
        dpad = _pad16(d)
        Tp = _pad16(T)
        Bmat = _make_B(t, dpad, Tp)
        biasv = jnp.tile(_pad_vec(bi, Tp), (NROT,)).reshape(1, NROT * Tp)
        gv = _pad_vec(bg, Tp).reshape(1, Tp)
        bv = _pad_vec(bb, Tp).reshape(1, Tp)
        gpad = g if g.shape[1] == dpad else jnp.pad(g, ((0, 0), (0, dpad - g.shape[1])))
        rows3 = _sc_gather(gpad, idx_by_dpad[dpad], dpad)
        g = _isc_layer(rows3, w3, Bmat, biasv, gv, bv, dpad, Tp, True)

    T, _, _, d = tl.shape
    dpad = _pad16(d)
    Tp = _pad16(T)
    Bl = _make_B(tl, dpad, Tp)
    biaslv = jnp.tile(_pad_vec(biasl, Tp), (NROT,)).reshape(1, NROT * Tp)
    zeros = jnp.zeros((1, Tp), jnp.float32)
    rows3l = _sc_gather(x, idx_by_dpad[dpad], dpad)
    ltab = _isc_layer(rows3l, w3, Bl, biaslv, zeros, zeros, dpad, Tp, False)

    return _final(g, ltab, w_out, b_out)


# speedup vs baseline: 11.6078x; 1.0169x over previous
import functools
import math

import jax
import jax.numpy as jnp
from jax import lax
from jax.experimental import pallas as pl
from jax.experimental.pallas import tpu as pltpu
from jax.experimental.pallas import tpu_sc as plsc

V = 6890
D = 544
R = 5
A = 8
K3 = 3
RA = R * A
C120 = K3 * RA
ROT_DELTA = 2
ROTS = tuple(range(0, A, ROT_DELTA))
NROT = len(ROTS)
INV_SQRT = 1.0 / math.sqrt(1.0 + 1e-3)

NW = 32
VP = 6912
VW = VP // NW
CH = 54
NC_V = VW // CH
NBUF = 6


def _pad16(n):
    return ((n + 15) // 16) * 16


def _sc_gather(table, idx_sc, dpad):
    P = 128 // dpad
    NQ = C120 // P
    NU = NQ * NC_V
    NG = NU // NBUF
    assert NG % 2 == 0 and NG * NBUF == NU
    mesh = plsc.VectorSubcoreMesh(core_axis_name="c", subcore_axis_name="s")

    @functools.partial(
        pl.kernel,
        mesh=mesh,
        out_type=jax.ShapeDtypeStruct((NQ, VP, 128), jnp.float32),
        scratch_types=[
            pltpu.VMEM((NU * P, CH), jnp.int32),
            pltpu.VMEM((2 * NBUF, P, CH, dpad), jnp.float32),
            pltpu.SemaphoreType.DMA((2 * NBUF,)),
            pltpu.SemaphoreType.DMA((2 * NBUF,)),
        ],
        compiler_params=pltpu.CompilerParams(use_tc_tiling_on_sc=False),
    )
    def gather_kernel(idx_hbm, table_hbm, out_hbm, idx_v, rows_v, gsem, osem):
        c = lax.axis_index("c")
        s = lax.axis_index("s")
        wid = s * 2 + c
        pltpu.sync_copy(idx_hbm.at[wid], idx_v)
        vbase0 = wid * VW

        def gathers(u, slot, wait):
            for h in range(P):
                cp = pltpu.make_async_copy(
                    table_hbm.at[idx_v.at[u * P + h]],
                    rows_v.at[slot, h],
                    gsem.at[slot],
                )
                if wait:
                    cp.wait()
                else:
                    cp.start()

        def scatter(u, slot, wait):
            q = u // NC_V
            vb = vbase0 + (u % NC_V) * CH
            for h in range(P):
                cp = pltpu.make_async_copy(
                    rows_v.at[slot, h],
                    out_hbm.at[q, pl.ds(vb, CH), pl.ds(h * dpad, dpad)],
                    osem.at[slot],
                )
                if wait:
                    cp.wait()
                else:
                    cp.start()

        for b in range(NBUF):
            gathers(b, b, False)

        def group(g, base):
            for b in range(NBUF):
                u = g * NBUF + b
                gathers(u, base + b, True)
                scatter(u, base + b, False)
            other = NBUF - base

            @pl.when(g >= 1)
            def _():
                for b in range(NBUF):
                    scatter((g - 1) * NBUF + b, other + b, True)

            @pl.when(g + 1 < NG)
            def _():
                for b in range(NBUF):
                    gathers((g + 1) * NBUF + b, other + b, False)

        def pair(g2, carry):
            group(2 * g2, 0)
            group(2 * g2 + 1, NBUF)
            return carry

        lax.fori_loop(0, NG // 2, pair, 0)
        for b in range(NBUF):
            scatter((NG - 1) * NBUF + b, NBUF + b, True)

    return gather_kernel(idx_sc, table)


def _down_body(x_ref, w_ref, b_ref, g_ref, bb_ref, o_ref):
    y = jnp.dot(x_ref[...], w_ref[...], preferred_element_type=jnp.float32)
    y = jnp.maximum(y + b_ref[...], 0.0)
    o_ref[...] = g_ref[...] * (y * INV_SQRT) + bb_ref[...]


def _down(signal, w_down, b_down, g, b):
    BV = 576
    grid = VP // BV
    return pl.pallas_call(
        _down_body,
        grid=(grid,),
        in_specs=[
            pl.BlockSpec((BV, D), lambda i: (i, 0)),
            pl.BlockSpec((D, 64), lambda i: (0, 0)),
            pl.BlockSpec((1, 64), lambda i: (0, 0)),
            pl.BlockSpec((1, 64), lambda i: (0, 0)),
            pl.BlockSpec((1, 64), lambda i: (0, 0)),
        ],
        out_specs=pl.BlockSpec((BV, 64), lambda i: (i, 0)),
        out_shape=jax.ShapeDtypeStruct((VP, 64), jnp.float32),
    )(signal, w_down, b_down.reshape(1, 64), g.reshape(1, 64), b.reshape(1, 64))


def _interp_blocks(rows_ref, w_ref, e_ref, dpad):
    P = 128 // dpad
    NQ = C120 // P
    NK = RA // P
    wrows = []
    for q in range(NQ):
        wb = jnp.dot(w_ref[:, q * P:(q + 1) * P], e_ref[...],
                     preferred_element_type=jnp.float32)
        wrows.append(wb * rows_ref[q])
    return [wrows[qp] + wrows[NK + qp] + wrows[2 * NK + qp]
            for qp in range(NK)]


def _isc_head(interp, B_ref, bias_ref, Tp):
    NK = len(interp)
    bv = interp[0].shape[0]
    acc = jnp.zeros((bv, NROT * Tp), jnp.float32)
    if NK % 2 == 0:
        for j in range(NK // 2):
            blk = jnp.concatenate([interp[2 * j], interp[2 * j + 1]], axis=1)
            acc = acc + jnp.dot(blk, B_ref[j], preferred_element_type=jnp.float32)
    else:
        for j in range(NK):
            acc = acc + jnp.dot(interp[j], B_ref[j], preferred_element_type=jnp.float32)
    acc = jnp.maximum(acc + bias_ref[...], 0.0)
    return jnp.maximum(
        jnp.maximum(acc[:, 0 * Tp:1 * Tp], acc[:, 1 * Tp:2 * Tp]),
        jnp.maximum(acc[:, 2 * Tp:3 * Tp], acc[:, 3 * Tp:4 * Tp]),
    )


def _isc_body(rows_ref, w_ref, e_ref, B_ref, bias_ref, g_ref, bb_ref, o_ref, *,
              dpad, Tp, bn):
    interp = _interp_blocks(rows_ref, w_ref, e_ref, dpad)
    m = _isc_head(interp, B_ref, bias_ref, Tp)
    if bn:
        m = g_ref[...] * (m * INV_SQRT) + bb_ref[...]
    o_ref[...] = m


def _isc_dual_body(rows_ref, w_ref, e_ref, B0_ref, bias0_ref, g0_ref, b0_ref,
                   Bl_ref, biasl_ref, o0_ref, ol_ref, *, dpad, Tp):
    interp = _interp_blocks(rows_ref, w_ref, e_ref, dpad)
    m0 = _isc_head(interp, B0_ref, bias0_ref, Tp)
    o0_ref[...] = g0_ref[...] * (m0 * INV_SQRT) + b0_ref[...]
    ol_ref[...] = _isc_head(interp, Bl_ref, biasl_ref, Tp)


def _isc_layer(rows3, w3, Bmat, biasv, g, bb, dpad, Tp, bn):
    BV = 216
    grid = VP // BV
    NQ = rows3.shape[0]
    P = 128 // dpad
    NK = RA // P
    NB = NK // 2 if NK % 2 == 0 else NK
    Bmat = Bmat.reshape(NB, -1, NROT * Tp)
    emat = jnp.repeat(jnp.eye(P, dtype=jnp.float32), dpad, axis=1)
    body = functools.partial(_isc_body, dpad=dpad, Tp=Tp, bn=bn)
    return pl.pallas_call(
        body,
        grid=(grid,),
        in_specs=[
            pl.BlockSpec((NQ, BV, 128), lambda i: (0, i, 0)),
            pl.BlockSpec((BV, C120), lambda i: (i, 0)),
            pl.BlockSpec((P, 128), lambda i: (0, 0)),
            pl.BlockSpec(Bmat.shape, lambda i: (0, 0, 0)),
            pl.BlockSpec((1, NROT * Tp), lambda i: (0, 0)),
            pl.BlockSpec((1, Tp), lambda i: (0, 0)),
            pl.BlockSpec((1, Tp), lambda i: (0, 0)),
        ],
        out_specs=pl.BlockSpec((BV, Tp), lambda i: (i, 0)),
        out_shape=jax.ShapeDtypeStruct((VP, Tp), jnp.float32),
    )(rows3, w3, emat, Bmat, biasv, g, bb)


def _isc_layer_dual(rows3, w3, B0, bias0v, g0, b0, Bl, biaslv, dpad, Tp):
    BV = 216
    grid = VP // BV
    NQ = rows3.shape[0]
    P = 128 // dpad
    NK = RA // P
    NB = NK // 2 if NK % 2 == 0 else NK
    B0 = B0.reshape(NB, -1, NROT * Tp)
    Bl = Bl.reshape(NB, -1, NROT * Tp)
    emat = jnp.repeat(jnp.eye(P, dtype=jnp.float32), dpad, axis=1)
    body = functools.partial(_isc_dual_body, dpad=dpad, Tp=Tp)
    return pl.pallas_call(
        body,
        grid=(grid,),
        in_specs=[
            pl.BlockSpec((NQ, BV, 128), lambda i: (0, i, 0)),
            pl.BlockSpec((BV, C120), lambda i: (i, 0)),
            pl.BlockSpec((P, 128), lambda i: (0, 0)),
            pl.BlockSpec(B0.shape, lambda i: (0, 0, 0)),
            pl.BlockSpec((1, NROT * Tp), lambda i: (0, 0)),
            pl.BlockSpec((1, Tp), lambda i: (0, 0)),
            pl.BlockSpec((1, Tp), lambda i: (0, 0)),
            pl.BlockSpec(Bl.shape, lambda i: (0, 0, 0)),
            pl.BlockSpec((1, NROT * Tp), lambda i: (0, 0)),
        ],
        out_specs=[
            pl.BlockSpec((BV, Tp), lambda i: (i, 0)),
            pl.BlockSpec((BV, Tp), lambda i: (i, 0)),
        ],
        out_shape=[
            jax.ShapeDtypeStruct((VP, Tp), jnp.float32),
            jax.ShapeDtypeStruct((VP, Tp), jnp.float32),
        ],
    )(rows3, w3, emat, B0, bias0v, g0, b0, Bl, biaslv)


def _final_body(g_ref, l_ref, w_ref, b_ref, o_ref):
    cat = jnp.concatenate([g_ref[:, :50], l_ref[:, :50]], axis=1)
    o_ref[...] = jnp.dot(cat, w_ref[...], preferred_element_type=jnp.float32) + b_ref[...]


def _final(gtab, ltab, w_out, b_out):
    BV = 512
    grid = (V + BV - 1) // BV
    return pl.pallas_call(
        _final_body,
        grid=(grid,),
        in_specs=[
            pl.BlockSpec((BV, 64), lambda i: (i, 0)),
            pl.BlockSpec((BV, 64), lambda i: (i, 0)),
            pl.BlockSpec((100, V), lambda i: (0, 0)),
            pl.BlockSpec((1, V), lambda i: (0, 0)),
        ],
        out_specs=pl.BlockSpec((BV, V), lambda i: (i, 0)),
        out_shape=jax.ShapeDtypeStruct((V, V), jnp.float32),
    )(gtab, ltab, w_out, b_out.reshape(1, V))


def _make_B(t, dpad, Tp):
    T, _, _, d = t.shape
    rolls = jnp.stack([jnp.roll(t, rot, axis=2) for rot in ROTS], axis=0)
    B = jnp.transpose(rolls, (2, 3, 4, 0, 1))
    B = jnp.pad(B, ((0, 0), (0, 0), (0, dpad - d), (0, 0), (0, Tp - T)))
    return B.reshape(RA, dpad, NROT * Tp)


def _pack_idx(idx_c, dpad):
    P = 128 // dpad
    NQ = C120 // P
    x = idx_c.reshape(NQ, P, NW, NC_V, CH)
    x = jnp.transpose(x, (2, 0, 3, 1, 4))
    return x.reshape(NW, NQ * NC_V * P, CH)


def _pad_vec(x, n):
    return jnp.pad(x, (0, n - x.shape[0]))


def kernel(signal, bc_idx, bc_w, w_down, b_down, bn_down_g, bn_down_b,
           t0, bias0, bn0_g, bn0_b, t1, bias1, bn1_g, bn1_b,
           t2, bias2, bn2_g, bn2_b, t3, bias3, bn3_g, bn3_b,
           t4, bias4, bn4_g, bn4_b, tl, biasl, w_out, b_out):
    idx_t = jnp.transpose(bc_idx, (3, 1, 2, 0)).reshape(C120, V)
    idx_c = jnp.pad(idx_t, ((0, 0), (0, VP - V)))
    idx_by_dpad = {dp: _pack_idx(idx_c, dp) for dp in (64, 32, 16)}

    w3 = jnp.transpose(bc_w, (3, 1, 2, 0)).reshape(C120, V).T
    w3 = jnp.pad(w3, ((0, VP - V), (0, 0)))

    x = _down(signal, w_down, b_down, bn_down_g, bn_down_b)

    layers = [
        (t0, bias0, bn0_g, bn0_b),
        (t1, bias1, bn1_g, bn1_b),
        (t2, bias2, bn2_g, bn2_b),
        (t3, bias3, bn3_g, bn3_b),
        (t4, bias4, bn4_g, bn4_b),
    ]
    t, bi, bg, bb = layers[0]
    Tp = _pad16(t.shape[0])
    rows3 = _sc_gather(x, idx_by_dpad[64], 64)
    g, ltab = _isc_layer_dual(
        rows3, w3,
        _make_B(t, 64, Tp),
        jnp.tile(_pad_vec(bi, Tp), (NROT,)).reshape(1, NROT * Tp),
        _pad_vec(bg, Tp).reshape(1, Tp), _pad_vec(bb, Tp).reshape(1, Tp),
        _make_B(tl, 64, Tp),
        jnp.tile(_pad_vec(biasl, Tp), (NROT,)).reshape(1, NROT * Tp),
        64, Tp)

    for t, bi, bg, bb in layers[1:]:
        T, _, _, d = t.shape
        dpad = _pad16(d)
        Tp = _pad16(T)
        Bmat = _make_B(t, dpad, Tp)
        biasv = jnp.tile(_pad_vec(bi, Tp), (NROT,)).reshape(1, NROT * Tp)
        gv = _pad_vec(bg, Tp).reshape(1, Tp)
        bv = _pad_vec(bb, Tp).reshape(1, Tp)
        gpad = g if g.shape[1] == dpad else jnp.pad(g, ((0, 0), (0, dpad - g.shape[1])))
        rows3 = _sc_gather(gpad, idx_by_dpad[dpad], dpad)
        g = _isc_layer(rows3, w3, Bmat, biasv, gv, bv, dpad, Tp, True)

    return _final(g, ltab, w_out, b_out)
